# Initial kernel scaffold; baseline (speedup 1.0000x reference)
import jax
import jax.numpy as jnp
from jax.experimental import pallas as pl


def kernel(x, Wq, bq, Wk, bk, A, edge_index):
    raise NotImplementedError("write your pallas kernel here")


# speedup vs baseline: 103.6936x; 103.6936x over previous
import jax
import jax.numpy as jnp
from jax import lax
from jax.experimental import pallas as pl
from jax.experimental.pallas import tpu as pltpu
from jax.experimental.pallas import tpu_sc as plsc

N = 10000
E = 320000
D = 128
H = 8
C = 16
HC = H * C
NPAD = 10240
ROWW = 144
NC = 2
NS = 16
NW = NC * NS
EPW = E // NW
CH = 128
NFULL = EPW // CH
TAIL = EPW - NFULL * CH
BN = 1024
RPT = NPAD // NS

_HIGH = lax.Precision.HIGHEST


def _tc1_body(x_ref, wq_ref, bq_ref, ab_ref, q_ref, qa_ref, gm_ref):
    xq = jnp.dot(x_ref[...], wq_ref[...], precision=_HIGH) + bq_ref[...]
    q_ref[...] = xq
    qa = jnp.dot(xq, ab_ref[...], precision=_HIGH)
    qa_ref[...] = qa
    bm = jnp.max(qa, axis=0, keepdims=True)
    i = pl.program_id(0)

    @pl.when(i == 0)
    def _():
        gm_ref[...] = bm

    @pl.when(i != 0)
    def _():
        gm_ref[...] = jnp.maximum(gm_ref[...], bm)


def _tc2_body(q_ref, qa_ref, gm_ref, p_ref, e8_ref, t_ref):
    qz = jnp.exp(qa_ref[...] - gm_ref[...])
    qzrep = jnp.dot(qz, p_ref[...], precision=_HIGH)
    t_ref[:, :HC] = q_ref[...] * qzrep
    t_ref[:, HC:ROWW] = jnp.dot(qz, e8_ref[...], precision=_HIGH)


def _tc3_body(a0_ref, a1_ref, p_ref, o_ref):
    s = a0_ref[...] + a1_ref[...]
    den = s[:, HC:HC + H]
    dexp = jnp.dot(den, p_ref[...], precision=_HIGH)
    o_ref[...] = jnp.maximum(s[:, :HC] / (dexp + 1e-16), 0.0)


def _sc_body(t_hbm, src_hbm, dst_hbm, z_hbm, out_hbm,
             sidx, didx, rows, sidx_t, didx_t, rows_t, accum, sem):
    c = lax.axis_index("c")
    s = lax.axis_index("s")
    base = pl.multiple_of((c * NS + s) * EPW, 8)
    rb = pl.multiple_of(s * RPT, 8)

    pltpu.sync_copy(z_hbm.at[pl.ds(rb, RPT)], accum.at[pl.ds(rb, RPT)])
    plsc.subcore_barrier()

    def body(j, carry):
        eb = pl.multiple_of(base + j * CH, 8)
        pltpu.sync_copy(src_hbm.at[pl.ds(eb, CH)], sidx)
        pltpu.sync_copy(dst_hbm.at[pl.ds(eb, CH)], didx)
        pltpu.async_copy(t_hbm.at[sidx], rows, sem).wait()
        pltpu.sync_copy(rows, accum.at[didx], add=True)
        return carry

    lax.fori_loop(0, NFULL, body, 0, unroll=False)

    eb = pl.multiple_of(base + NFULL * CH, 8)
    pltpu.sync_copy(src_hbm.at[pl.ds(eb, TAIL)], sidx_t)
    pltpu.sync_copy(dst_hbm.at[pl.ds(eb, TAIL)], didx_t)
    pltpu.async_copy(t_hbm.at[sidx_t], rows_t, sem).wait()
    pltpu.sync_copy(rows_t, accum.at[didx_t], add=True)

    plsc.subcore_barrier()
    pltpu.sync_copy(accum.at[pl.ds(rb, RPT)],
                    out_hbm.at[pl.ds(c * NPAD + rb, RPT)])


def kernel(x, Wq, bq, Wk, bk, A, edge_index):
    del Wk, bk
    f32 = jnp.float32
    x_pad = jnp.pad(x, ((0, NPAD - N), (0, 0)))
    ab = (A.T[:, :, None] * jnp.eye(H, dtype=f32)[:, None, :]).reshape(HC, H)
    p_exp = jnp.kron(jnp.eye(H, dtype=f32), jnp.ones((1, C), f32))
    e8 = jnp.concatenate([jnp.eye(H, dtype=f32),
                          jnp.zeros((H, ROWW - HC - H), f32)], axis=1)
    bq2 = bq.reshape(1, HC)
    src = edge_index[0]
    dst = edge_index[1]
    zrows = jnp.zeros((NPAD, ROWW), f32)

    grid = NPAD // BN
    q, qa, gm = pl.pallas_call(
        _tc1_body,
        grid=(grid,),
        in_specs=[
            pl.BlockSpec((BN, D), lambda i: (i, 0)),
            pl.BlockSpec((D, HC), lambda i: (0, 0)),
            pl.BlockSpec((1, HC), lambda i: (0, 0)),
            pl.BlockSpec((HC, H), lambda i: (0, 0)),
        ],
        out_specs=[
            pl.BlockSpec((BN, HC), lambda i: (i, 0)),
            pl.BlockSpec((BN, H), lambda i: (i, 0)),
            pl.BlockSpec((1, H), lambda i: (0, 0)),
        ],
        out_shape=[
            jax.ShapeDtypeStruct((NPAD, HC), f32),
            jax.ShapeDtypeStruct((NPAD, H), f32),
            jax.ShapeDtypeStruct((1, H), f32),
        ],
    )(x_pad, Wq, bq2, ab)

    t_tab = pl.pallas_call(
        _tc2_body,
        grid=(grid,),
        in_specs=[
            pl.BlockSpec((BN, HC), lambda i: (i, 0)),
            pl.BlockSpec((BN, H), lambda i: (i, 0)),
            pl.BlockSpec((1, H), lambda i: (0, 0)),
            pl.BlockSpec((H, HC), lambda i: (0, 0)),
            pl.BlockSpec((H, ROWW - HC), lambda i: (0, 0)),
        ],
        out_specs=pl.BlockSpec((BN, ROWW), lambda i: (i, 0)),
        out_shape=jax.ShapeDtypeStruct((NPAD, ROWW), f32),
    )(q, qa, gm, p_exp, e8)

    mesh = plsc.VectorSubcoreMesh(core_axis_name="c", subcore_axis_name="s",
                                  num_cores=NC, num_subcores=NS)
    acc = pl.kernel(
        _sc_body,
        out_type=jax.ShapeDtypeStruct((NC * NPAD, ROWW), f32),
        mesh=mesh,
        scratch_types=[
            pltpu.VMEM((CH,), jnp.int32),
            pltpu.VMEM((CH,), jnp.int32),
            pltpu.VMEM((CH, ROWW), f32),
            pltpu.VMEM((TAIL,), jnp.int32),
            pltpu.VMEM((TAIL,), jnp.int32),
            pltpu.VMEM((TAIL, ROWW), f32),
            pltpu.VMEM_SHARED((NPAD, ROWW), f32),
            pltpu.SemaphoreType.DMA,
        ],
        compiler_params=pltpu.CompilerParams(use_tc_tiling_on_sc=False),
    )(t_tab, src, dst, zrows)

    out_full = pl.pallas_call(
        _tc3_body,
        grid=(grid,),
        in_specs=[
            pl.BlockSpec((BN, ROWW), lambda i: (i, 0)),
            pl.BlockSpec((BN, ROWW), lambda i: (i + grid, 0)),
            pl.BlockSpec((H, HC), lambda i: (0, 0)),
        ],
        out_specs=pl.BlockSpec((BN, HC), lambda i: (i, 0)),
        out_shape=jax.ShapeDtypeStruct((NPAD, HC), f32),
    )(acc, acc, p_exp)

    return out_full[:N]
